# Initial kernel scaffold; baseline (speedup 1.0000x reference)
import jax
import jax.numpy as jnp
from jax.experimental import pallas as pl


def kernel(input_BX):
    raise NotImplementedError("write your pallas kernel here")


# speedup vs baseline: 8.4146x; 8.4146x over previous
import jax
import jax.numpy as jnp
import numpy as np
from jax import lax
from jax.experimental import pallas as pl
from jax.experimental.pallas import tpu as pltpu

_B = 128
_X = 32768
_K = 64
_TOPK = _K * _B
_MIN32 = np.int32(-2147483648)
_RC = 8
_NCH = _B // _RC


def _sortable_u(x):
    s = lax.bitcast_convert_type(x, jnp.int32)
    return s ^ ((s >> 31) | _MIN32)


def _count_chunked(x_ref, pred_fn):
    def chunk(c, acc):
        r0 = c * _RC
        x = x_ref[pl.ds(r0, _RC), :]
        ku = _sortable_u(x)
        row = lax.broadcasted_iota(jnp.int32, (_RC, _X), 0) + r0
        col = lax.broadcasted_iota(jnp.int32, (_RC, _X), 1)
        idx = row * _X + col
        return acc + jnp.sum(pred_fn(ku, idx).astype(jnp.int32))

    return lax.fori_loop(0, _NCH, chunk, jnp.int32(0))


def _body(x_ref, o_ref, st_ref):
    i = pl.program_id(0)

    @pl.when(i == 0)
    def _init():
        st_ref[0] = 0
        st_ref[1] = 0
        st_ref[2] = 0
        st_ref[3] = 0

    @pl.when(i < 32)
    def _value_bit():
        t = 31 - i
        p = st_ref[0]
        r = st_ref[1]
        cand = lax.shift_right_logical(p, t) | 1
        c = _count_chunked(
            x_ref, lambda ku, idx: lax.shift_right_logical(ku, t) == cand)
        take = (r + c) >= _TOPK
        st_ref[0] = jnp.where(take, p | lax.shift_left(jnp.int32(1), t), p)
        st_ref[1] = jnp.where(take, r, r + c)

        @pl.when(t == 0)
        def _finish_value():
            st_ref[3] = _TOPK - st_ref[1]

    @pl.when((i >= 32) & (i < 54))
    def _index_bit():
        t = 53 - i
        tt = st_ref[0]
        q = st_ref[2]
        nr = st_ref[3]
        qs = lax.shift_right_logical(q, t)
        c0 = _count_chunked(
            x_ref,
            lambda ku, idx: (ku == tt)
            & (lax.shift_right_logical(idx, t) == qs))
        go_right = c0 < nr
        st_ref[2] = jnp.where(go_right, q | lax.shift_left(jnp.int32(1), t), q)
        st_ref[3] = jnp.where(go_right, nr - c0, nr)

    @pl.when(i == 54)
    def _mask():
        tt = st_ref[0]
        q = st_ref[2]
        ttm = tt ^ _MIN32

        def chunk(c, _):
            r0 = c * _RC
            x = x_ref[pl.ds(r0, _RC), :]
            ku = _sortable_u(x)
            row = lax.broadcasted_iota(jnp.int32, (_RC, _X), 0) + r0
            col = lax.broadcasted_iota(jnp.int32, (_RC, _X), 1)
            idx = row * _X + col
            keep = ((ku ^ _MIN32) > ttm) | ((ku == tt) & (idx <= q))
            o_ref[pl.ds(r0, _RC), :] = x * keep.astype(jnp.float32)
            return _

        lax.fori_loop(0, _NCH, chunk, jnp.int32(0))


def kernel(input_BX):
    return pl.pallas_call(
        _body,
        grid=(55,),
        in_specs=[pl.BlockSpec((_B, _X), lambda i: (0, 0))],
        out_specs=pl.BlockSpec((_B, _X), lambda i: (0, 0)),
        out_shape=jax.ShapeDtypeStruct((_B, _X), jnp.float32),
        scratch_shapes=[pltpu.SMEM((4,), jnp.int32)],
    )(input_BX)
